# baseline (speedup 1.0000x reference)
import functools

import jax
import jax.numpy as jnp
from jax import lax
from jax.experimental import pallas as pl
from jax.experimental.pallas import tpu as pltpu
from jax.experimental.pallas import tpu_sc as plsc

_B = 4096
_L = 200
_EMB = 32
_NBUF = 4


def _make_gather(nbuf: int):
    info = plsc.get_sparse_core_info()
    nc, ns = info.num_cores, info.num_subcores
    nw = nc * ns
    n_units = _L * (_B // 128)
    u_per_w = n_units // nw
    n_groups = u_per_w // nbuf
    assert n_groups * nbuf == u_per_w

    mesh = plsc.VectorSubcoreMesh(core_axis_name="c", subcore_axis_name="s")

    @functools.partial(
        pl.kernel,
        mesh=mesh,
        out_type=jax.ShapeDtypeStruct((_L, _EMB // 8, _B // 128, 8, 128), jnp.float32),
        scratch_types=[
            pltpu.VMEM((u_per_w, 128), jnp.int32),
            [pltpu.VMEM((128, _EMB), jnp.float32) for _ in range(nbuf)],
            [pltpu.VMEM((_EMB, 128), jnp.float32) for _ in range(nbuf)],
            [pltpu.SemaphoreType.DMA for _ in range(nbuf)],
            [pltpu.SemaphoreType.DMA for _ in range(nbuf)],
        ],
        compiler_params=pltpu.CompilerParams(
            use_tc_tiling_on_sc=False, needs_layout_passes=False
        ),
    )
    def gather_kernel(idx_hbm, table_hbm, out_hbm, idx_v, rows, tbuf, gsem, ssem):
        wid = lax.axis_index("s") * nc + lax.axis_index("c")
        ubase = wid * u_per_w
        pltpu.sync_copy(idx_hbm.at[pl.ds(ubase, u_per_w)], idx_v)

        riota = lax.iota(jnp.int32, 16)
        rowis = [riota + blk * 16 for blk in range(8)]

        def fire_gather(i, b):
            pltpu.async_copy(table_hbm.at[idx_v.at[i]], rows[b], gsem[b])

        def transpose_unit(b):
            @plsc.parallel_loop(0, _EMB, unroll=4)
            def _(e0):
                erot = jnp.bitwise_and(e0 + riota, _EMB - 1)
                for blk in range(8):
                    v = plsc.load_gather(rows[b], [rowis[blk], erot])
                    plsc.store_scatter(tbuf[b], [erot, rowis[blk]], v)

        def store_unit(i, b):
            u = ubase + i
            l = u // (_B // 128)
            tb = u % (_B // 128)
            for te in range(_EMB // 8):
                pltpu.async_copy(
                    tbuf[b].at[pl.ds(te * 8, 8)], out_hbm.at[l, te, tb], ssem[b]
                )

        def wait_store(i, b):
            u = ubase + i
            l = u // (_B // 128)
            tb = u % (_B // 128)
            for te in range(_EMB // 8):
                pltpu.make_async_copy(
                    tbuf[b].at[pl.ds(te * 8, 8)], out_hbm.at[l, te, tb], ssem[b]
                ).wait()

        for b in range(nbuf):
            fire_gather(b, b)

        def group(g, carry):
            for b in range(nbuf):
                i = g * nbuf + b
                pltpu.make_async_copy(
                    table_hbm.at[idx_v.at[i]], rows[b], gsem[b]
                ).wait()

                @pl.when(g > 0)
                def _():
                    wait_store(i - nbuf, b)

                transpose_unit(b)
                store_unit(i, b)
                j = i + nbuf

                @pl.when(j < u_per_w)
                def _():
                    fire_gather(j, b)

            return carry

        lax.fori_loop(0, n_groups, group, 0)
        for b in range(nbuf):
            wait_store(u_per_w - nbuf + b, b)

    return gather_kernel


_V = 1000000


def _make_table_transpose(nbuf: int):
    info = plsc.get_sparse_core_info()
    nc, ns = info.num_cores, info.num_subcores
    nw = nc * ns
    n_full = _V // 128
    tail_w = _V - n_full * 128
    n_units = n_full + 1
    slots = -(-n_units // nw)
    n_groups = -(-slots // nbuf)

    mesh = plsc.VectorSubcoreMesh(core_axis_name="c", subcore_axis_name="s")

    @functools.partial(
        pl.kernel,
        mesh=mesh,
        out_type=jax.ShapeDtypeStruct((_V, _EMB), jnp.float32),
        scratch_types=[
            [pltpu.VMEM((_EMB, 128), jnp.float32) for _ in range(nbuf)],
            [pltpu.VMEM((128, _EMB), jnp.float32) for _ in range(nbuf)],
            pltpu.VMEM((_EMB, tail_w), jnp.float32),
            pltpu.VMEM((tail_w, _EMB), jnp.float32),
            [pltpu.SemaphoreType.DMA for _ in range(nbuf)],
            [pltpu.SemaphoreType.DMA for _ in range(nbuf)],
        ],
        compiler_params=pltpu.CompilerParams(
            use_tc_tiling_on_sc=False, needs_layout_passes=False
        ),
    )
    def transpose_kernel(tt_hbm, out_hbm, va, ta, vtail, ttail, gsem, ssem):
        wid = lax.axis_index("s") * nc + lax.axis_index("c")

        riota = lax.iota(jnp.int32, 16)
        rowis = [riota + blk * 16 for blk in range(8)]

        def unit_of(s):
            return wid + s * nw

        def fire_load(s, b):
            pltpu.async_copy(
                tt_hbm.at[:, pl.ds(unit_of(s) * 128, 128)], va[b], gsem[b]
            )

        def wait_load(s, b):
            pltpu.make_async_copy(
                tt_hbm.at[:, pl.ds(unit_of(s) * 128, 128)], va[b], gsem[b]
            ).wait()

        def transpose_block(src, dst, nblk):
            @plsc.parallel_loop(0, _EMB, unroll=4)
            def _(e0):
                erot = jnp.bitwise_and(e0 + riota, _EMB - 1)
                for blk in range(nblk):
                    v = plsc.load_gather(src, [erot, rowis[blk]])
                    plsc.store_scatter(dst, [rowis[blk], erot], v)

        def wait_store(s, b):
            pltpu.make_async_copy(
                out_hbm.at[pl.ds(unit_of(s) * 128, 128)], ta[b], ssem[b]
            ).wait()

        for b in range(nbuf):
            fire_load(b, b)

        def group(g, carry):
            for b in range(nbuf):
                s = g * nbuf + b
                u = unit_of(s)

                @pl.when(u < n_full)
                def _():
                    wait_load(s, b)

                    @pl.when(s >= nbuf)
                    def _():
                        wait_store(s - nbuf, b)

                    transpose_block(va[b], ta[b], 8)
                    pltpu.async_copy(
                        ta[b], out_hbm.at[pl.ds(u * 128, 128)], ssem[b]
                    )
                    nxt = s + nbuf

                    @pl.when(unit_of(nxt) < n_full)
                    def _():
                        fire_load(nxt, b)

                @pl.when(u == n_full)
                def _():
                    pltpu.sync_copy(tt_hbm.at[:, pl.ds(u * 128, tail_w)], vtail)
                    transpose_block(vtail, ttail, tail_w // 16)
                    pltpu.sync_copy(ttail, out_hbm.at[pl.ds(u * 128, tail_w)])

            return carry

        lax.fori_loop(0, n_groups, group, 0)
        for s in range(n_groups * nbuf - 2 * nbuf, n_groups * nbuf):
            @pl.when(
                (unit_of(s) < n_full) & (unit_of(s + nbuf) >= n_full)
            )
            def _():
                wait_store(s, s % nbuf)

    return transpose_kernel


def kernel(word_inputs, table):
    idx_units = (
        word_inputs.T.astype(jnp.int32).reshape(_L, _B // 128, 128).reshape(-1, 128)
    )
    tbl = _make_table_transpose(_NBUF)(table.T)
    out5d = _make_gather(_NBUF)(idx_units, tbl)
    return out5d.transpose(2, 4, 0, 1, 3).reshape(_B, _L, _EMB)


# speedup vs baseline: 4.6430x; 4.6430x over previous
import functools

import jax
import jax.numpy as jnp
from jax import lax
from jax.experimental import pallas as pl
from jax.experimental.pallas import tpu as pltpu
from jax.experimental.pallas import tpu_sc as plsc

_B = 4096
_L = 200
_EMB = 32
_NBUF = 4


def _make_gather(nbuf: int):
    info = plsc.get_sparse_core_info()
    nc, ns = info.num_cores, info.num_subcores
    nw = nc * ns
    n_units = _L * (_B // 128)
    u_per_w = n_units // nw
    n_groups = u_per_w // nbuf
    assert n_groups * nbuf == u_per_w

    mesh = plsc.VectorSubcoreMesh(core_axis_name="c", subcore_axis_name="s")

    @functools.partial(
        pl.kernel,
        mesh=mesh,
        out_type=jax.ShapeDtypeStruct((_L, _EMB // 8, _B // 128, 8, 128), jnp.float32),
        scratch_types=[
            pltpu.VMEM((u_per_w, 128), jnp.int32),
            [pltpu.VMEM((128, _EMB), jnp.float32) for _ in range(nbuf)],
            [pltpu.VMEM((_EMB, 128), jnp.float32) for _ in range(nbuf)],
            [pltpu.SemaphoreType.DMA for _ in range(nbuf)],
            [pltpu.SemaphoreType.DMA for _ in range(nbuf)],
        ],
        compiler_params=pltpu.CompilerParams(
            use_tc_tiling_on_sc=False, needs_layout_passes=False
        ),
    )
    def gather_kernel(idx_hbm, table_hbm, out_hbm, idx_v, rows, tbuf, gsem, ssem):
        wid = lax.axis_index("s") * nc + lax.axis_index("c")
        ubase = wid * u_per_w
        pltpu.sync_copy(idx_hbm.at[pl.ds(ubase, u_per_w)], idx_v)

        riota = lax.iota(jnp.int32, 16)
        rowis = [riota + blk * 16 for blk in range(8)]

        def fire_gather(i, b):
            pltpu.async_copy(table_hbm.at[idx_v.at[i]], rows[b], gsem[b])

        def transpose_unit(b):
            @plsc.parallel_loop(0, _EMB, unroll=4)
            def _(e0):
                erot = jnp.bitwise_and(e0 + riota, _EMB - 1)
                for blk in range(8):
                    v = plsc.load_gather(rows[b], [rowis[blk], erot])
                    plsc.store_scatter(tbuf[b], [erot, rowis[blk]], v)

        def store_unit(i, b):
            u = ubase + i
            l = u // (_B // 128)
            tb = u % (_B // 128)
            for te in range(_EMB // 8):
                pltpu.async_copy(
                    tbuf[b].at[pl.ds(te * 8, 8)], out_hbm.at[l, te, tb], ssem[b]
                )

        def wait_store(i, b):
            u = ubase + i
            l = u // (_B // 128)
            tb = u % (_B // 128)
            for te in range(_EMB // 8):
                pltpu.make_async_copy(
                    tbuf[b].at[pl.ds(te * 8, 8)], out_hbm.at[l, te, tb], ssem[b]
                ).wait()

        for b in range(nbuf):
            fire_gather(b, b)

        def group(g, carry):
            for b in range(nbuf):
                i = g * nbuf + b
                pltpu.make_async_copy(
                    table_hbm.at[idx_v.at[i]], rows[b], gsem[b]
                ).wait()

                @pl.when(g > 0)
                def _():
                    wait_store(i - nbuf, b)

                transpose_unit(b)
                store_unit(i, b)
                j = i + nbuf

                @pl.when(j < u_per_w)
                def _():
                    fire_gather(j, b)

            return carry

        lax.fori_loop(0, n_groups, group, 0)
        for b in range(nbuf):
            wait_store(u_per_w - nbuf + b, b)

    return gather_kernel


def kernel(word_inputs, table):
    idx_units = (
        word_inputs.T.astype(jnp.int32).reshape(_L, _B // 128, 128).reshape(-1, 128)
    )
    out5d = _make_gather(_NBUF)(idx_units, table)
    return out5d.transpose(2, 4, 0, 1, 3).reshape(_B, _L, _EMB)
